# baseline (speedup 1.0000x reference)
import functools

import jax
import jax.numpy as jnp
from jax import lax
from jax.experimental import pallas as pl
from jax.experimental.pallas import tpu as pltpu
from jax.experimental.pallas import tpu_sc as plsc

_B, _N, _CIN = 4, 8192, 64
_M = 1024
_RADII = (0.1, 0.2, 0.4)
_KS = (16, 32, 64)
_D = 80
_SUB, _LANE = 64, 128
_EPS = 1e-5


def _fps_body(xyz_ref, out_ref, dist_ref):
    x = xyz_ref[0]
    y = xyz_ref[1]
    z = xyz_ref[2]
    shp = (_B, _SUB, _LANE)
    dist_ref[...] = jnp.full(shp, 1e10, jnp.float32)
    lin = (lax.broadcasted_iota(jnp.int32, shp, 1) * _LANE
           + lax.broadcasted_iota(jnp.int32, shp, 2))
    mshp = (_B, _M // _LANE, _LANE)
    miota = (lax.broadcasted_iota(jnp.int32, mshp, 1) * _LANE
             + lax.broadcasted_iota(jnp.int32, mshp, 2))

    def _sum_bk(v):
        return jnp.sum(jnp.sum(v, axis=2, keepdims=True), axis=1, keepdims=True)

    def body(i, st):
        f, ax, ay, az = st
        sel = lin == f
        zf = jnp.float32(0.0)
        cx = _sum_bk(jnp.where(sel, x, zf))
        cy = _sum_bk(jnp.where(sel, y, zf))
        cz = _sum_bk(jnp.where(sel, z, zf))
        hit = miota == i
        ax = ax + jnp.where(hit, cx, zf)
        ay = ay + jnp.where(hit, cy, zf)
        az = az + jnp.where(hit, cz, zf)
        dx = x - cx
        dy = y - cy
        dz = z - cz
        d = dx * dx + dy * dy + dz * dz
        nd = jnp.minimum(dist_ref[...], d)
        dist_ref[...] = nd
        mx = jnp.max(jnp.max(nd, axis=2, keepdims=True), axis=1, keepdims=True)
        cand = jnp.where(nd == mx, lin, jnp.int32(2**30))
        f = jnp.min(jnp.min(cand, axis=2, keepdims=True), axis=1, keepdims=True)
        return f, ax, ay, az

    zacc = jnp.zeros(mshp, jnp.float32)
    _, ax, ay, az = lax.fori_loop(
        0, _M, body, (jnp.zeros((_B, 1, 1), jnp.int32), zacc, zacc, zacc))
    out_ref[...] = jnp.stack([ax, ay, az])


def _fps(xyz):
    xyz_r = xyz.transpose(2, 0, 1).reshape(3, _B, _SUB, _LANE)
    out = pl.pallas_call(
        _fps_body,
        out_shape=jax.ShapeDtypeStruct((3, _B, _M // _LANE, _LANE), jnp.float32),
        scratch_shapes=[pltpu.VMEM((_B, _SUB, _LANE), jnp.float32)],
    )(xyz_r)
    return out.reshape(3, _B, _M)


def _group_sc(xyz2, nxyz2, table2):
    info = plsc.get_sparse_core_info()
    nc, ns = info.num_cores, info.num_subcores
    nw = nc * ns
    cpw = (_B * _M) // nw
    wpb = nw // _B
    r2 = tuple(r * r for r in _RADII)
    pad1, pad2, pad3 = _KS[0] + 16, _KS[1] + 16, _KS[2] + 16
    nchunk = _N // 16
    mesh = plsc.VectorSubcoreMesh(core_axis_name="c", subcore_axis_name="s")

    nbuf = 4

    @functools.partial(
        pl.kernel, mesh=mesh,
        compiler_params=pltpu.CompilerParams(needs_layout_passes=False,
                                             use_tc_tiling_on_sc=False),
        out_type=[jax.ShapeDtypeStruct((_B * _M * k, _D), jnp.float32)
                  for k in _KS],
        scratch_types=(
            [pltpu.VMEM((3 * _N,), jnp.float32),
             pltpu.VMEM((3 * (cpw + 16),), jnp.float32),
             pltpu.VMEM((cpw * pad1,), jnp.int32),
             pltpu.VMEM((cpw * pad2,), jnp.int32),
             pltpu.VMEM((cpw * pad3,), jnp.int32)]
            + [pltpu.VMEM((k, _D), jnp.float32)
               for _b in range(nbuf) for k in _KS]
            + [pltpu.SemaphoreType.DMA] * (2 * 3 * nbuf)
        ))
    def grouped(xyz_hbm, nxyz_hbm, tab_hbm, g1, g2, g3,
                pts, cen, i1, i2, i3, *rest):
        bufs = [rest[b * 3:b * 3 + 3] for b in range(nbuf)]
        gsems = [rest[3 * nbuf + b * 3:3 * nbuf + b * 3 + 3]
                 for b in range(nbuf)]
        ssems = [rest[6 * nbuf + b * 3:6 * nbuf + b * 3 + 3]
                 for b in range(nbuf)]
        gouts = (g1, g2, g3)
        idxs = (i1, i2, i3)
        pads = (pad1, pad2, pad3)
        wid = lax.axis_index("s") * nc + lax.axis_index("c")
        b = wid // wpb
        m0 = (wid % wpb) * cpw
        ibase = b * _N
        for c in range(3):
            pltpu.sync_copy(xyz_hbm.at[pl.ds((b * 3 + c) * _N, _N)],
                            pts.at[pl.ds(c * _N, _N)])
            pltpu.sync_copy(nxyz_hbm.at[pl.ds((b * 3 + c) * _M + m0, cpw)],
                            cen.at[pl.ds(c * (cpw + 16), cpw)])

        lanes = lax.iota(jnp.int32, 16)

        def per_centroid(i, carry):
            del carry
            cxv = cen[pl.ds(i, 16)]
            cyv = cen[pl.ds((cpw + 16) + i, 16)]
            czv = cen[pl.ds(2 * (cpw + 16) + i, 16)]
            cx, cy, cz = cxv[0], cyv[0], czv[0]

            @plsc.parallel_loop(0, nchunk, 1, unroll=8,
                                carry=(jnp.int32(0), jnp.int32(0),
                                       jnp.int32(0)))
            def counts(ch, st):
                c1, c2, c3 = st
                off = ch * 16
                px = pts[pl.ds(off, 16)]
                py = pts[pl.ds(_N + off, 16)]
                pz = pts[pl.ds(2 * _N + off, 16)]
                dx = px - cx
                dy = py - cy
                dz = pz - cz
                d = dx * dx + dy * dy + dz * dz
                iv = (off + ibase) + lanes
                m1 = d <= r2[0]
                m2 = d <= r2[1]
                m3 = d <= r2[2]
                plsc.store_compressed(
                    i1.at[pl.ds(i * pad1 + jnp.minimum(c1, _KS[0]), 16)],
                    iv, mask=m1)
                plsc.store_compressed(
                    i2.at[pl.ds(i * pad2 + jnp.minimum(c2, _KS[1]), 16)],
                    iv, mask=m2)
                plsc.store_compressed(
                    i3.at[pl.ds(i * pad3 + jnp.minimum(c3, _KS[2]), 16)],
                    iv, mask=m3)
                c1 = c1 + plsc.all_reduce_population_count(m1)[0]
                c2 = c2 + plsc.all_reduce_population_count(m2)[0]
                c3 = c3 + plsc.all_reduce_population_count(m3)[0]
                return c1, c2, c3

            c1, c2, c3 = counts

            for idx_t, kk, pad, cnt in (
                    (i1, _KS[0], pad1, c1), (i2, _KS[1], pad2, c2),
                    (i3, _KS[2], pad3, c3)):
                n_ok = jnp.minimum(cnt, kk)
                base = i * pad
                first = idx_t[pl.ds(base, 16)][0]
                for j in range(kk // 16):
                    pos = lanes + (j * 16)
                    cur = idx_t[pl.ds(base + j * 16, 16)]
                    idx_t[pl.ds(base + j * 16, 16)] = jnp.where(
                        pos < n_ok, cur, first)
            return 0

        lax.fori_loop(0, cpw, per_centroid, 0)

        def fire_gather(slot, i):
            for st in range(3):
                pltpu.make_async_copy(
                    tab_hbm.at[idxs[st].at[pl.ds(i * pads[st], _KS[st])]],
                    bufs[slot][st], gsems[slot][st]).start()

        def wait_gather_fire_store(slot, i):
            for st in range(3):
                pltpu.make_async_copy(
                    tab_hbm.at[idxs[st].at[pl.ds(i * pads[st], _KS[st])]],
                    bufs[slot][st], gsems[slot][st]).wait()
                row0 = (wid * cpw + i) * _KS[st]
                pltpu.make_async_copy(
                    bufs[slot][st],
                    gouts[st].at[pl.ds(row0, _KS[st])],
                    ssems[slot][st]).start()

        def wait_store(slot, i):
            for st in range(3):
                row0 = (wid * cpw + i) * _KS[st]
                pltpu.make_async_copy(
                    bufs[slot][st],
                    gouts[st].at[pl.ds(row0, _KS[st])],
                    ssems[slot][st]).wait()

        for bslot in range(nbuf):
            fire_gather(bslot, bslot)

        def pipe_body(g, carry):
            del carry
            i0 = g * nbuf
            for bslot in range(nbuf):
                wait_gather_fire_store(bslot, i0 + bslot)
            for bslot in range(nbuf):
                ip = i0 + bslot + nbuf

                @pl.when(ip < cpw)
                def _():
                    wait_store(bslot, ip - nbuf)
                    fire_gather(bslot, ip)
            return 0

        lax.fori_loop(0, cpw // nbuf, pipe_body, 0)
        for bslot in range(nbuf):
            wait_store(bslot, cpw - nbuf + bslot)

    return [o.reshape(_B * _M, k, _D)
            for o, k in zip(grouped(xyz2, nxyz2, table2), _KS)]


_INV_SQRT2 = 0.7071067811865476


def _gelu(x):
    return x * 0.5 * (1.0 + lax.erf(x * _INV_SQRT2))


def _stats_update(st_ref, y3):
    s = jnp.sum(jnp.sum(y3, axis=0), axis=0)
    ss = jnp.sum(jnp.sum(y3 * y3, axis=0), axis=0)
    st = jnp.stack([s, ss])

    @pl.when(pl.program_id(0) == 0)
    def _():
        st_ref[...] = st

    @pl.when(pl.program_id(0) > 0)
    def _():
        st_ref[...] = st_ref[...] + st


def _l1_body(x_ref, nx_ref, w_ref, wx_ref, y_ref, st_ref):
    cm, k, d = x_ref.shape
    y = jnp.dot(x_ref[...].reshape(cm * k, d), w_ref[...],
                preferred_element_type=jnp.float32)
    corr = jnp.dot(nx_ref[...], wx_ref[...],
                   preferred_element_type=jnp.float32)
    y3 = y.reshape(cm, k, -1) - corr[:, None, :]
    y_ref[...] = y3
    _stats_update(st_ref, y3)


def _mid_body(x_ref, sc_ref, sh_ref, w_ref, y_ref, st_ref):
    cm, k, c = x_ref.shape
    h = _gelu(x_ref[...] * sc_ref[...] + sh_ref[...])
    y3 = jnp.dot(h.reshape(cm * k, c), w_ref[...],
                 preferred_element_type=jnp.float32).reshape(cm, k, -1)
    y_ref[...] = y3
    _stats_update(st_ref, y3)


def _last_body(x_ref, sc_ref, sh_ref, o_ref):
    k = x_ref.shape[1]
    h = _gelu(x_ref[...] * sc_ref[...] + sh_ref[...])
    m = h[:, 0, :]
    for j in range(1, k):
        m = jnp.maximum(m, h[:, j, :])
    o_ref[...] = m


def _bn_coeffs(st, gamma, beta, count):
    mean = st[0] / count
    var = st[1] / count - mean * mean
    inv = lax.rsqrt(var + _EPS)
    scale = gamma * inv
    shift = beta - mean * scale
    return scale.reshape(1, 1, -1), shift.reshape(1, 1, -1)


def _mlp_scale(g, nx_pad, layers, k):
    bm = _B * _M
    cm = 2048 // k
    grid = (bm // cm,)
    count = jnp.float32(bm * k)

    (w1, g1, b1), (w2, g2, b2), (w3, g3, b3) = layers
    c1, c2, c3 = w1.shape[0], w2.shape[0], w3.shape[0]
    w1a = jnp.pad(w1.T, ((0, _D - w1.shape[1]), (0, 0)))
    w1x = jnp.pad(w1[:, :3].T, ((0, 5), (0, 0)))

    y1, st1 = pl.pallas_call(
        _l1_body,
        grid=grid,
        in_specs=[
            pl.BlockSpec((cm, k, _D), lambda i: (i, 0, 0)),
            pl.BlockSpec((cm, 8), lambda i: (i, 0)),
            pl.BlockSpec((_D, c1), lambda i: (0, 0)),
            pl.BlockSpec((8, c1), lambda i: (0, 0)),
        ],
        out_specs=[
            pl.BlockSpec((cm, k, c1), lambda i: (i, 0, 0)),
            pl.BlockSpec((2, c1), lambda i: (0, 0)),
        ],
        out_shape=[
            jax.ShapeDtypeStruct((bm, k, c1), jnp.float32),
            jax.ShapeDtypeStruct((2, c1), jnp.float32),
        ],
    )(g, nx_pad, w1a, w1x)

    def mid(x, st, gam, bet, w, cin, cout):
        sc, sh = _bn_coeffs(st, gam, bet, count)
        return pl.pallas_call(
            _mid_body,
            grid=grid,
            in_specs=[
                pl.BlockSpec((cm, k, cin), lambda i: (i, 0, 0)),
                pl.BlockSpec((1, 1, cin), lambda i: (0, 0, 0)),
                pl.BlockSpec((1, 1, cin), lambda i: (0, 0, 0)),
                pl.BlockSpec((cin, cout), lambda i: (0, 0)),
            ],
            out_specs=[
                pl.BlockSpec((cm, k, cout), lambda i: (i, 0, 0)),
                pl.BlockSpec((2, cout), lambda i: (0, 0)),
            ],
            out_shape=[
                jax.ShapeDtypeStruct((bm, k, cout), jnp.float32),
                jax.ShapeDtypeStruct((2, cout), jnp.float32),
            ],
        )(x, sc, sh, w.T)

    y2, st2 = mid(y1, st1, g1, b1, w2, c1, c2)
    y3, st3 = mid(y2, st2, g2, b2, w3, c2, c3)

    sc, sh = _bn_coeffs(st3, g3, b3, count)
    out = pl.pallas_call(
        _last_body,
        grid=grid,
        in_specs=[
            pl.BlockSpec((cm, k, c3), lambda i: (i, 0, 0)),
            pl.BlockSpec((1, 1, c3), lambda i: (0, 0, 0)),
            pl.BlockSpec((1, 1, c3), lambda i: (0, 0, 0)),
        ],
        out_specs=pl.BlockSpec((cm, c3), lambda i: (i, 0)),
        out_shape=jax.ShapeDtypeStruct((bm, c3), jnp.float32),
    )(y3, sc, sh)
    return out


def kernel(xyz, features, params):
    new_xyz_t = _fps(xyz)
    new_xyz = new_xyz_t.transpose(1, 2, 0)

    xyz2 = xyz.transpose(0, 2, 1).reshape(_B * 3 * _N)
    nxyz2 = new_xyz_t.transpose(1, 0, 2).reshape(_B * 3 * _M)
    table2 = jnp.pad(
        jnp.concatenate([xyz, features], axis=-1).reshape(_B * _N, 3 + _CIN),
        ((0, 0), (0, _D - 3 - _CIN)))
    gs = _group_sc(xyz2, nxyz2, table2)

    nx_pad = jnp.pad(new_xyz.reshape(_B * _M, 3), ((0, 0), (0, 5)))
    outs = []
    for g, layers, k in zip(gs, params, _KS):
        o = _mlp_scale(g, nx_pad, layers, k)
        outs.append(o.reshape(_B, _M, -1).transpose(0, 2, 1))
    new_features = jnp.concatenate(outs, axis=1)
    return (new_xyz, new_features)


# speedup vs baseline: 1.3112x; 1.3112x over previous
import functools

import jax
import jax.numpy as jnp
from jax import lax
from jax.experimental import pallas as pl
from jax.experimental.pallas import tpu as pltpu
from jax.experimental.pallas import tpu_sc as plsc

_B, _N, _CIN = 4, 8192, 64
_M = 1024
_RADII = (0.1, 0.2, 0.4)
_KS = (16, 32, 64)
_D = 80
_SUB, _LANE = 64, 128
_EPS = 1e-5


def _fps_body(xyz_ref, out_ref, dist_ref):
    x = xyz_ref[0]
    y = xyz_ref[1]
    z = xyz_ref[2]
    shp = (_B, _SUB, _LANE)
    dist_ref[...] = jnp.full(shp, 1e10, jnp.float32)
    lin = (lax.broadcasted_iota(jnp.int32, shp, 1) * _LANE
           + lax.broadcasted_iota(jnp.int32, shp, 2))
    mshp = (_B, _M // _LANE, _LANE)
    miota = (lax.broadcasted_iota(jnp.int32, mshp, 1) * _LANE
             + lax.broadcasted_iota(jnp.int32, mshp, 2))

    def _sum_bk(v):
        return jnp.sum(jnp.sum(v, axis=2, keepdims=True), axis=1, keepdims=True)

    def body(i, st):
        f, ax, ay, az = st
        sel = lin == f
        zf = jnp.float32(0.0)
        cx = _sum_bk(jnp.where(sel, x, zf))
        cy = _sum_bk(jnp.where(sel, y, zf))
        cz = _sum_bk(jnp.where(sel, z, zf))
        hit = miota == i
        ax = ax + jnp.where(hit, cx, zf)
        ay = ay + jnp.where(hit, cy, zf)
        az = az + jnp.where(hit, cz, zf)
        dx = x - cx
        dy = y - cy
        dz = z - cz
        d = dx * dx + dy * dy + dz * dz
        nd = jnp.minimum(dist_ref[...], d)
        dist_ref[...] = nd
        mx = jnp.max(jnp.max(nd, axis=2, keepdims=True), axis=1, keepdims=True)
        cand = jnp.where(nd == mx, lin, jnp.int32(2**30))
        f = jnp.min(jnp.min(cand, axis=2, keepdims=True), axis=1, keepdims=True)
        return f, ax, ay, az

    zacc = jnp.zeros(mshp, jnp.float32)
    _, ax, ay, az = lax.fori_loop(
        0, _M, body, (jnp.zeros((_B, 1, 1), jnp.int32), zacc, zacc, zacc))
    out_ref[...] = jnp.stack([ax, ay, az])


def _fps(xyz):
    xyz_r = xyz.transpose(2, 0, 1).reshape(3, _B, _SUB, _LANE)
    out = pl.pallas_call(
        _fps_body,
        out_shape=jax.ShapeDtypeStruct((3, _B, _M // _LANE, _LANE), jnp.float32),
        scratch_shapes=[pltpu.VMEM((_B, _SUB, _LANE), jnp.float32)],
    )(xyz_r)
    return out.reshape(3, _B, _M)


def _group_sc(xyz2, nxyz2, table2):
    info = plsc.get_sparse_core_info()
    nc, ns = info.num_cores, info.num_subcores
    nw = nc * ns
    cpw = (_B * _M) // nw
    wpb = nw // _B
    r2 = tuple(r * r for r in _RADII)
    pad1, pad2, pad3 = _KS[0] + 16, _KS[1] + 16, _KS[2] + 16
    nchunk = _N // 16
    mesh = plsc.VectorSubcoreMesh(core_axis_name="c", subcore_axis_name="s")

    nbuf = 4

    @functools.partial(
        pl.kernel, mesh=mesh,
        compiler_params=pltpu.CompilerParams(needs_layout_passes=False,
                                             use_tc_tiling_on_sc=False),
        out_type=[jax.ShapeDtypeStruct((_B * _M * k, _D), jnp.float32)
                  for k in _KS],
        scratch_types=(
            [pltpu.VMEM((3 * _N,), jnp.float32),
             pltpu.VMEM((3 * (cpw + 16),), jnp.float32),
             pltpu.VMEM((cpw * pad1,), jnp.int32),
             pltpu.VMEM((cpw * pad2,), jnp.int32),
             pltpu.VMEM((cpw * pad3,), jnp.int32)]
            + [pltpu.VMEM((k, _D), jnp.float32)
               for _b in range(nbuf) for k in _KS]
            + [pltpu.SemaphoreType.DMA] * (2 * 3 * nbuf)
        ))
    def grouped(xyz_hbm, nxyz_hbm, tab_hbm, g1, g2, g3,
                pts, cen, i1, i2, i3, *rest):
        bufs = [rest[b * 3:b * 3 + 3] for b in range(nbuf)]
        gsems = [rest[3 * nbuf + b * 3:3 * nbuf + b * 3 + 3]
                 for b in range(nbuf)]
        ssems = [rest[6 * nbuf + b * 3:6 * nbuf + b * 3 + 3]
                 for b in range(nbuf)]
        gouts = (g1, g2, g3)
        idxs = (i1, i2, i3)
        pads = (pad1, pad2, pad3)
        wid = lax.axis_index("s") * nc + lax.axis_index("c")
        b = wid // wpb
        m0 = (wid % wpb) * cpw
        ibase = b * _N
        for c in range(3):
            pltpu.sync_copy(xyz_hbm.at[pl.ds((b * 3 + c) * _N, _N)],
                            pts.at[pl.ds(c * _N, _N)])
            pltpu.sync_copy(nxyz_hbm.at[pl.ds((b * 3 + c) * _M + m0, cpw)],
                            cen.at[pl.ds(c * (cpw + 16), cpw)])

        lanes = lax.iota(jnp.int32, 16)

        def per_centroid(i, carry):
            del carry
            cxv = cen[pl.ds(i, 16)]
            cyv = cen[pl.ds((cpw + 16) + i, 16)]
            czv = cen[pl.ds(2 * (cpw + 16) + i, 16)]
            cx, cy, cz = cxv[0], cyv[0], czv[0]

            @plsc.parallel_loop(0, nchunk, 1, unroll=4,
                                carry=(jnp.int32(0), jnp.int32(0),
                                       jnp.int32(0)))
            def counts(ch, st):
                c1, c2, c3 = st
                off = ch * 16
                px = pts[pl.ds(off, 16)]
                py = pts[pl.ds(_N + off, 16)]
                pz = pts[pl.ds(2 * _N + off, 16)]
                dx = px - cx
                dy = py - cy
                dz = pz - cz
                d = dx * dx + dy * dy + dz * dz
                iv = (off + ibase) + lanes
                m1 = d <= r2[0]
                m2 = d <= r2[1]
                m3 = d <= r2[2]
                plsc.store_compressed(
                    i1.at[pl.ds(i * pad1 + jnp.minimum(c1, _KS[0]), 16)],
                    iv, mask=m1)
                plsc.store_compressed(
                    i2.at[pl.ds(i * pad2 + jnp.minimum(c2, _KS[1]), 16)],
                    iv, mask=m2)
                plsc.store_compressed(
                    i3.at[pl.ds(i * pad3 + jnp.minimum(c3, _KS[2]), 16)],
                    iv, mask=m3)
                c1 = c1 + plsc.all_reduce_population_count(m1)[0]
                c2 = c2 + plsc.all_reduce_population_count(m2)[0]
                c3 = c3 + plsc.all_reduce_population_count(m3)[0]
                return c1, c2, c3

            c1, c2, c3 = counts

            for idx_t, kk, pad, cnt in (
                    (i1, _KS[0], pad1, c1), (i2, _KS[1], pad2, c2),
                    (i3, _KS[2], pad3, c3)):
                n_ok = jnp.minimum(cnt, kk)
                base = i * pad
                first = idx_t[pl.ds(base, 16)][0]
                for j in range(kk // 16):
                    pos = lanes + (j * 16)
                    cur = idx_t[pl.ds(base + j * 16, 16)]
                    idx_t[pl.ds(base + j * 16, 16)] = jnp.where(
                        pos < n_ok, cur, first)
            return 0

        lax.fori_loop(0, cpw, per_centroid, 0)

        def fire_gather(slot, i):
            for st in range(3):
                pltpu.make_async_copy(
                    tab_hbm.at[idxs[st].at[pl.ds(i * pads[st], _KS[st])]],
                    bufs[slot][st], gsems[slot][st]).start()

        def wait_gather_fire_store(slot, i):
            for st in range(3):
                pltpu.make_async_copy(
                    tab_hbm.at[idxs[st].at[pl.ds(i * pads[st], _KS[st])]],
                    bufs[slot][st], gsems[slot][st]).wait()
                row0 = (wid * cpw + i) * _KS[st]
                pltpu.make_async_copy(
                    bufs[slot][st],
                    gouts[st].at[pl.ds(row0, _KS[st])],
                    ssems[slot][st]).start()

        def wait_store(slot, i):
            for st in range(3):
                row0 = (wid * cpw + i) * _KS[st]
                pltpu.make_async_copy(
                    bufs[slot][st],
                    gouts[st].at[pl.ds(row0, _KS[st])],
                    ssems[slot][st]).wait()

        for bslot in range(nbuf):
            fire_gather(bslot, bslot)

        def pipe_body(g, carry):
            del carry
            i0 = g * nbuf
            for bslot in range(nbuf):
                wait_gather_fire_store(bslot, i0 + bslot)
            for bslot in range(nbuf):
                ip = i0 + bslot + nbuf

                @pl.when(ip < cpw)
                def _():
                    wait_store(bslot, ip - nbuf)
                    fire_gather(bslot, ip)
            return 0

        lax.fori_loop(0, cpw // nbuf, pipe_body, 0)
        for bslot in range(nbuf):
            wait_store(bslot, cpw - nbuf + bslot)

    return [o.reshape(_B * _M, k, _D)
            for o, k in zip(grouped(xyz2, nxyz2, table2), _KS)]


_INV_SQRT2 = 0.7071067811865476


def _gelu(x):
    return x * 0.5 * (1.0 + lax.erf(x * _INV_SQRT2))


def _stats_update(st_ref, y3):
    s = jnp.sum(jnp.sum(y3, axis=0), axis=0)
    ss = jnp.sum(jnp.sum(y3 * y3, axis=0), axis=0)
    st = jnp.stack([s, ss])

    @pl.when(pl.program_id(0) == 0)
    def _():
        st_ref[...] = st

    @pl.when(pl.program_id(0) > 0)
    def _():
        st_ref[...] = st_ref[...] + st


def _l1_body(x_ref, nx_ref, w_ref, wx_ref, y_ref, st_ref):
    cm, k, d = x_ref.shape
    y = jnp.dot(x_ref[...].reshape(cm * k, d), w_ref[...],
                preferred_element_type=jnp.float32)
    corr = jnp.dot(nx_ref[...], wx_ref[...],
                   preferred_element_type=jnp.float32)
    y3 = y.reshape(cm, k, -1) - corr[:, None, :]
    y_ref[...] = y3
    _stats_update(st_ref, y3)


def _mid_body(x_ref, sc_ref, sh_ref, w_ref, y_ref, st_ref):
    cm, k, c = x_ref.shape
    h = _gelu(x_ref[...] * sc_ref[...] + sh_ref[...])
    y3 = jnp.dot(h.reshape(cm * k, c), w_ref[...],
                 preferred_element_type=jnp.float32).reshape(cm, k, -1)
    y_ref[...] = y3
    _stats_update(st_ref, y3)


def _last_body(x_ref, sc_ref, sh_ref, o_ref):
    k = x_ref.shape[1]
    h = _gelu(x_ref[...] * sc_ref[...] + sh_ref[...])
    m = h[:, 0, :]
    for j in range(1, k):
        m = jnp.maximum(m, h[:, j, :])
    o_ref[...] = m


def _bn_coeffs(st, gamma, beta, count):
    mean = st[0] / count
    var = st[1] / count - mean * mean
    inv = lax.rsqrt(var + _EPS)
    scale = gamma * inv
    shift = beta - mean * scale
    return scale.reshape(1, 1, -1), shift.reshape(1, 1, -1)


def _mlp_scale(g, nx_pad, layers, k):
    bm = _B * _M
    cm = 2048 // k
    grid = (bm // cm,)
    count = jnp.float32(bm * k)

    (w1, g1, b1), (w2, g2, b2), (w3, g3, b3) = layers
    c1, c2, c3 = w1.shape[0], w2.shape[0], w3.shape[0]
    w1a = jnp.pad(w1.T, ((0, _D - w1.shape[1]), (0, 0)))
    w1x = jnp.pad(w1[:, :3].T, ((0, 5), (0, 0)))

    y1, st1 = pl.pallas_call(
        _l1_body,
        grid=grid,
        in_specs=[
            pl.BlockSpec((cm, k, _D), lambda i: (i, 0, 0)),
            pl.BlockSpec((cm, 8), lambda i: (i, 0)),
            pl.BlockSpec((_D, c1), lambda i: (0, 0)),
            pl.BlockSpec((8, c1), lambda i: (0, 0)),
        ],
        out_specs=[
            pl.BlockSpec((cm, k, c1), lambda i: (i, 0, 0)),
            pl.BlockSpec((2, c1), lambda i: (0, 0)),
        ],
        out_shape=[
            jax.ShapeDtypeStruct((bm, k, c1), jnp.float32),
            jax.ShapeDtypeStruct((2, c1), jnp.float32),
        ],
    )(g, nx_pad, w1a, w1x)

    def mid(x, st, gam, bet, w, cin, cout):
        sc, sh = _bn_coeffs(st, gam, bet, count)
        return pl.pallas_call(
            _mid_body,
            grid=grid,
            in_specs=[
                pl.BlockSpec((cm, k, cin), lambda i: (i, 0, 0)),
                pl.BlockSpec((1, 1, cin), lambda i: (0, 0, 0)),
                pl.BlockSpec((1, 1, cin), lambda i: (0, 0, 0)),
                pl.BlockSpec((cin, cout), lambda i: (0, 0)),
            ],
            out_specs=[
                pl.BlockSpec((cm, k, cout), lambda i: (i, 0, 0)),
                pl.BlockSpec((2, cout), lambda i: (0, 0)),
            ],
            out_shape=[
                jax.ShapeDtypeStruct((bm, k, cout), jnp.float32),
                jax.ShapeDtypeStruct((2, cout), jnp.float32),
            ],
        )(x, sc, sh, w.T)

    y2, st2 = mid(y1, st1, g1, b1, w2, c1, c2)
    y3, st3 = mid(y2, st2, g2, b2, w3, c2, c3)

    sc, sh = _bn_coeffs(st3, g3, b3, count)
    out = pl.pallas_call(
        _last_body,
        grid=grid,
        in_specs=[
            pl.BlockSpec((cm, k, c3), lambda i: (i, 0, 0)),
            pl.BlockSpec((1, 1, c3), lambda i: (0, 0, 0)),
            pl.BlockSpec((1, 1, c3), lambda i: (0, 0, 0)),
        ],
        out_specs=pl.BlockSpec((cm, c3), lambda i: (i, 0)),
        out_shape=jax.ShapeDtypeStruct((bm, c3), jnp.float32),
    )(y3, sc, sh)
    return out


def kernel(xyz, features, params):
    new_xyz_t = _fps(xyz)
    new_xyz = new_xyz_t.transpose(1, 2, 0)

    xyz2 = xyz.transpose(0, 2, 1).reshape(_B * 3 * _N)
    nxyz2 = new_xyz_t.transpose(1, 0, 2).reshape(_B * 3 * _M)
    table2 = jnp.pad(
        jnp.concatenate([xyz, features], axis=-1).reshape(_B * _N, 3 + _CIN),
        ((0, 0), (0, _D - 3 - _CIN)))
    gs = _group_sc(xyz2, nxyz2, table2)

    nx_pad = jnp.pad(new_xyz.reshape(_B * _M, 3), ((0, 0), (0, 5)))
    outs = []
    for g, layers, k in zip(gs, params, _KS):
        o = _mlp_scale(g, nx_pad, layers, k)
        outs.append(o.reshape(_B, _M, -1).transpose(0, 2, 1))
    new_features = jnp.concatenate(outs, axis=1)
    return (new_xyz, new_features)
